# Initial kernel scaffold; baseline (speedup 1.0000x reference)
import jax
import jax.numpy as jnp
from jax.experimental import pallas as pl


def kernel(x, pos, edge_index, batch, node_type, y, uvp_dim, sigma, params):
    raise NotImplementedError("write your pallas kernel here")


# speedup vs baseline: 3.3459x; 3.3459x over previous
import functools

import jax
import jax.numpy as jnp
from jax import lax
from jax.experimental import pallas as pl
from jax.experimental.pallas import tpu as pltpu
from jax.experimental.pallas import tpu_sc as plsc

PHI = 3
NB = 2000
EB = 8000


def _dot(a, b):
    return jax.lax.dot_general(a, b, (((1,), (0,)), ((), ())),
                               preferred_element_type=jnp.float32)


def _xmin_body(xp_ref, bt_ref, out_ref, acc_ref):
    i = pl.program_id(0)
    big = jnp.float32(3.4e38)

    @pl.when(i == 0)
    def _():
        acc_ref[...] = jnp.full(acc_ref.shape, big, jnp.float32)

    xp = xp_ref[...]
    bt = bt_ref[...]
    rows = []
    for b in range(out_ref.shape[0]):
        m = bt == b
        rows.append(jnp.min(jnp.where(m, xp, big), axis=0, keepdims=True))
    acc_ref[...] = jnp.minimum(acc_ref[...], jnp.concatenate(rows, axis=0))

    @pl.when(i == pl.num_programs(0) - 1)
    def _():
        out_ref[...] = acc_ref[...]


def _node_pre_body(x_ref, pos_ref, bt_ref, xminp_ref, shift_ref, stdp_ref,
                   wne_ref, bne_ref, ws_ref, wr_ref,
                   t_ref, hn_ref, a_ref, b_ref):
    x = x_ref[...]
    bt = bt_ref[...]
    xminp = xminp_ref[...]
    sh = jnp.zeros_like(x)
    for b in range(xminp.shape[0]):
        sh = jnp.where(bt == b, xminp[b], sh)
    xn = (x - sh - shift_ref[...]) / stdp_ref[...]
    hn = jax.nn.relu(_dot(xn, wne_ref[...]) + bne_ref[...])
    t_ref[...] = jnp.concatenate(
        [xn, pos_ref[...], jnp.zeros((x.shape[0], 6), jnp.float32)], axis=1)
    hn_ref[...] = hn
    a_ref[...] = _dot(hn, ws_ref[...])
    b_ref[...] = _dot(hn, wr_ref[...])


def _edge_enc_body(ts_ref, tr_ref, wp_ref, wn_ref, bee_ref, he_ref):
    d = ts_ref[...] - tr_ref[...]
    dx = d[:, 8:9]
    dy = d[:, 9:10]
    nrm = jnp.sqrt(dx * dx + dy * dy)
    he_ref[...] = jax.nn.relu(_dot(d, wp_ref[...]) + nrm * wn_ref[...]
                              + bee_ref[...])


def _edge_mlp_body(he_ref, ga_ref, gb_ref, w0_ref, be_ref, out_ref):
    he = he_ref[...]
    out_ref[...] = he + jax.nn.relu(
        _dot(he, w0_ref[...]) + ga_ref[...] + gb_ref[...] + be_ref[...])


def _node_mlp_proj_body(hn_ref, agg_ref, wn0_ref, wn1_ref, bn_ref,
                        ws_ref, wr_ref, hn1_ref, a_ref, b_ref):
    hn = hn_ref[...]
    hn1 = hn + jax.nn.relu(_dot(hn, wn0_ref[...]) + _dot(agg_ref[...], wn1_ref[...])
                           + bn_ref[...])
    hn1_ref[...] = hn1
    a_ref[...] = _dot(hn1, ws_ref[...])
    b_ref[...] = _dot(hn1, wr_ref[...])


def _node_mlp_dec_body(hn_ref, agg_ref, wn0_ref, wn1_ref, bn_ref,
                       wdec_ref, bdec_ref, nt_ref, bt_ref, y8_ref, sc8_ref,
                       out_ref):
    hn = hn_ref[...]
    hn1 = hn + jax.nn.relu(_dot(hn, wn0_ref[...]) + _dot(agg_ref[...], wn1_ref[...])
                           + bn_ref[...])
    uvp = _dot(hn1, wdec_ref[...]) + bdec_ref[...]
    uvp = jnp.tanh(uvp / 10.0) * 10.0
    nt = nt_ref[...]
    m_d = nt <= 3
    m_p = nt == 2
    uv_part = jnp.where(m_d, y8_ref[...], uvp)
    p_part = jnp.where(m_p, 0.0, uvp)
    li = lax.broadcasted_iota(jnp.int32, uvp.shape, 1)
    res = jnp.where(li < 2, uv_part, p_part)
    bt = bt_ref[...]
    sc8 = sc8_ref[...]
    ssel = jnp.zeros_like(res)
    for b in range(sc8.shape[0]):
        ssel = jnp.where(bt == b, sc8[b], ssel)
    out_ref[...] = res * ssel


def _full(shape):
    return pl.BlockSpec(shape, lambda i: (0, 0))


def _blk(rows, cols):
    return pl.BlockSpec((rows, cols), lambda i: (i, 0))


def _sc_gather2(tab1, idx1, tab2, idx2, D):
    E = idx1.shape[0]
    C = 1000
    per = E // 16
    nch = per // C
    mesh = plsc.VectorSubcoreMesh(core_axis_name="c", subcore_axis_name="s")

    @functools.partial(
        pl.kernel,
        out_type=(jax.ShapeDtypeStruct((E, D), jnp.float32),
                  jax.ShapeDtypeStruct((E, D), jnp.float32)),
        mesh=mesh,
        scratch_types=[pltpu.VMEM((C,), jnp.int32),
                       pltpu.VMEM((C, D), jnp.float32),
                       pltpu.SemaphoreType.DMA],
        compiler_params=pltpu.CompilerParams(use_tc_tiling_on_sc=False),
    )
    def k(t1, i1, t2, i2, o1, o2, iv, rv, sem):
        cid = lax.axis_index("c")
        sid = lax.axis_index("s")

        def run(t, i, o):
            def body(j, _):
                base = sid * per + j * C
                pltpu.sync_copy(i.at[pl.ds(base, C)], iv)
                pltpu.async_copy(t.at[iv], rv, sem).wait()
                pltpu.sync_copy(rv, o.at[pl.ds(base, C)])
                return 0
            lax.fori_loop(0, nch, body, 0)

        @pl.when(cid == 0)
        def _():
            run(t1, i1, o1)

        @pl.when(cid == 1)
        def _():
            run(t2, i2, o2)

    return k(tab1, idx1, tab2, idx2)


def _sc_segsum(he, r_idx, zeros_half, n_nodes):
    E, H = he.shape
    HH = H // 2
    C = 400
    per = E // 16
    nch = per // C
    rpt = n_nodes // 16
    mesh = plsc.VectorSubcoreMesh(core_axis_name="c", subcore_axis_name="s")

    @functools.partial(
        pl.kernel,
        out_type=jax.ShapeDtypeStruct((n_nodes, H), jnp.float32),
        mesh=mesh,
        scratch_types=[pltpu.VMEM((C,), jnp.int32),
                       pltpu.VMEM((C, HH), jnp.float32),
                       pltpu.VMEM_SHARED((n_nodes, HH), jnp.float32),
                       pltpu.SemaphoreType.DMA],
        compiler_params=pltpu.CompilerParams(use_tc_tiling_on_sc=False),
    )
    def k(he_h, r_h, z_h, out_h, iv, rv, acc, sem):
        cid = lax.axis_index("c")
        sid = lax.axis_index("s")
        pltpu.sync_copy(z_h.at[pl.ds(sid * rpt, rpt)],
                        acc.at[pl.ds(sid * rpt, rpt)])
        plsc.subcore_barrier()

        def body(j, _):
            base = sid * per + j * C
            pltpu.sync_copy(r_h.at[pl.ds(base, C)], iv)
            pltpu.sync_copy(he_h.at[pl.ds(base, C), pl.ds(cid * HH, HH)], rv)
            pltpu.sync_copy(rv, acc.at[iv], add=True)
            return 0
        lax.fori_loop(0, nch, body, 0)
        plsc.subcore_barrier()
        pltpu.sync_copy(acc.at[pl.ds(sid * rpt, rpt)],
                        out_h.at[pl.ds(sid * rpt, rpt), pl.ds(cid * HH, HH)])

    return k(he, r_idx, zeros_half)


def kernel(x, pos, edge_index, batch, node_type, y, uvp_dim, sigma, params):
    N, DIN = x.shape
    E = edge_index.shape[1]
    B = uvp_dim.shape[0]
    H = params["ne"][0].shape[1]
    s = edge_index[0]
    r = edge_index[1]
    bt2 = batch[:, None]
    nt2 = node_type[:, None]
    ng = N // NB
    eg = E // EB
    f32 = jnp.float32

    xmin = pl.pallas_call(
        _xmin_body,
        grid=(ng,),
        in_specs=[_blk(NB, PHI), _blk(NB, 1)],
        out_specs=pl.BlockSpec((B, PHI), lambda i: (0, 0)),
        out_shape=jax.ShapeDtypeStruct((B, PHI), f32),
        scratch_shapes=[pltpu.VMEM((B, PHI), f32)],
    )(x[:, :PHI], bt2)

    xminp = jnp.pad(xmin, ((0, 0), (0, DIN - PHI)))
    shift = jnp.pad(params["norm_mean"], (PHI, 0))[None]
    stdp = jnp.pad(params["norm_std"], (PHI, 0), constant_values=1.0)[None]
    wne, bne = params["ne"][0], params["ne"][1][None]
    wee, bee = params["ee"][0], params["ee"][1][None]
    wp = jnp.pad(wee[:DIN + 2], ((0, 16 - (DIN + 2)), (0, 0)))
    wn_row = wee[DIN + 2:DIN + 3]
    we0 = [params["e%d" % l][0][:H] for l in range(2)]
    wes = [params["e%d" % l][0][H:2 * H] for l in range(2)]
    wer = [params["e%d" % l][0][2 * H:] for l in range(2)]
    bel = [params["e%d" % l][1][None] for l in range(2)]
    wn0 = [params["n%d" % l][0][:H] for l in range(2)]
    wn1 = [params["n%d" % l][0][H:] for l in range(2)]
    bnl = [params["n%d" % l][1][None] for l in range(2)]
    wdec = jnp.pad(params["dec"][0], ((0, 0), (0, 8 - 3)))
    bdec = jnp.pad(params["dec"][1], (0, 8 - 3))[None]
    y8 = jnp.pad(y, ((0, 0), (0, 8 - 3)))
    sc8 = jnp.pad(uvp_dim * sigma, ((0, 0), (0, 8 - 3)))

    t, hn, a, b = pl.pallas_call(
        _node_pre_body,
        grid=(ng,),
        in_specs=[_blk(NB, DIN), _blk(NB, 2), _blk(NB, 1), _full((B, DIN)),
                  _full((1, DIN)), _full((1, DIN)), _full((DIN, H)),
                  _full((1, H)), _full((H, H)), _full((H, H))],
        out_specs=[_blk(NB, 16), _blk(NB, H), _blk(NB, H), _blk(NB, H)],
        out_shape=[jax.ShapeDtypeStruct((N, 16), f32),
                   jax.ShapeDtypeStruct((N, H), f32),
                   jax.ShapeDtypeStruct((N, H), f32),
                   jax.ShapeDtypeStruct((N, H), f32)],
    )(x, pos, bt2, xminp, shift, stdp, wne, bne, wes[0], wer[0])

    ts, tr = _sc_gather2(t, s, t, r, 16)
    he = pl.pallas_call(
        _edge_enc_body,
        grid=(eg,),
        in_specs=[_blk(EB, 16), _blk(EB, 16), _full((16, H)), _full((1, H)),
                  _full((1, H))],
        out_specs=_blk(EB, H),
        out_shape=jax.ShapeDtypeStruct((E, H), f32),
    )(ts, tr, wp, wn_row, bee)

    zeros_half = jnp.zeros((N, H // 2), f32)
    out = None
    for l in range(2):
        ga, gb = _sc_gather2(a, s, b, r, H)
        he = pl.pallas_call(
            _edge_mlp_body,
            grid=(eg,),
            in_specs=[_blk(EB, H), _blk(EB, H), _blk(EB, H), _full((H, H)),
                      _full((1, H))],
            out_specs=_blk(EB, H),
            out_shape=jax.ShapeDtypeStruct((E, H), f32),
        )(he, ga, gb, we0[l], bel[l])
        agg = _sc_segsum(he, r, zeros_half, N)
        if l == 0:
            hn, a, b = pl.pallas_call(
                _node_mlp_proj_body,
                grid=(ng,),
                in_specs=[_blk(NB, H), _blk(NB, H), _full((H, H)),
                          _full((H, H)), _full((1, H)), _full((H, H)),
                          _full((H, H))],
                out_specs=[_blk(NB, H), _blk(NB, H), _blk(NB, H)],
                out_shape=[jax.ShapeDtypeStruct((N, H), f32),
                           jax.ShapeDtypeStruct((N, H), f32),
                           jax.ShapeDtypeStruct((N, H), f32)],
            )(hn, agg, wn0[l], wn1[l], bnl[l], wes[1], wer[1])
        else:
            out = pl.pallas_call(
                _node_mlp_dec_body,
                grid=(ng,),
                in_specs=[_blk(NB, H), _blk(NB, H), _full((H, H)),
                          _full((H, H)), _full((1, H)), _full((H, 8)),
                          _full((1, 8)), _blk(NB, 1), _blk(NB, 1),
                          _blk(NB, 8), _full((B, 8))],
                out_specs=_blk(NB, 8),
                out_shape=jax.ShapeDtypeStruct((N, 8), f32),
            )(hn, agg, wn0[l], wn1[l], bnl[l], wdec, bdec, nt2, bt2, y8, sc8)

    return out[:, :3]


# speedup vs baseline: 6.2575x; 1.8702x over previous
import functools

import jax
import jax.numpy as jnp
from jax import lax
from jax.experimental import pallas as pl
from jax.experimental.pallas import tpu as pltpu
from jax.experimental.pallas import tpu_sc as plsc

PHI = 3
NB = 2000
EB = 8000


def _dot(a, b):
    return jax.lax.dot_general(a, b, (((1,), (0,)), ((), ())),
                               preferred_element_type=jnp.float32)


def _xmin_body(xp_ref, bt_ref, out_ref, acc_ref):
    i = pl.program_id(0)
    big = jnp.float32(3.4e38)

    @pl.when(i == 0)
    def _():
        acc_ref[...] = jnp.full(acc_ref.shape, big, jnp.float32)

    xp = xp_ref[...]
    bt = bt_ref[...]
    rows = []
    for b in range(out_ref.shape[0]):
        m = bt == b
        rows.append(jnp.min(jnp.where(m, xp, big), axis=0, keepdims=True))
    acc_ref[...] = jnp.minimum(acc_ref[...], jnp.concatenate(rows, axis=0))

    @pl.when(i == pl.num_programs(0) - 1)
    def _():
        out_ref[...] = acc_ref[...]


def _node_pre_body(x_ref, pos_ref, bt_ref, xminp_ref, shift_ref, stdp_ref,
                   wne_ref, bne_ref, ws_ref, wr_ref,
                   t_ref, hn_ref, a_ref, b_ref):
    x = x_ref[...]
    bt = bt_ref[...]
    xminp = xminp_ref[...]
    sh = jnp.zeros_like(x)
    for b in range(xminp.shape[0]):
        sh = jnp.where(bt == b, xminp[b], sh)
    xn = (x - sh - shift_ref[...]) / stdp_ref[...]
    hn = jax.nn.relu(_dot(xn, wne_ref[...]) + bne_ref[...])
    t_ref[...] = jnp.concatenate(
        [xn, pos_ref[...], jnp.zeros((x.shape[0], 6), jnp.float32)], axis=1)
    hn_ref[...] = hn
    a_ref[...] = _dot(hn, ws_ref[...])
    b_ref[...] = _dot(hn, wr_ref[...])


def _edge_l0_body(tp_ref, g_ref, wp_ref, wn_ref, bee_ref, w0_ref, be_ref,
                  out_ref):
    tp = tp_ref[...]
    d = tp[:, :16] - tp[:, 16:32]
    dx = d[:, 8:9]
    dy = d[:, 9:10]
    nrm = jnp.sqrt(dx * dx + dy * dy)
    he0 = jax.nn.relu(_dot(d, wp_ref[...]) + nrm * wn_ref[...] + bee_ref[...])
    g = g_ref[...]
    he1 = he0 + jax.nn.relu(
        _dot(he0, w0_ref[...]) + g[:, :64] + g[:, 64:] + be_ref[...])
    out_ref[...] = jnp.concatenate([he1, jnp.zeros_like(he1)], axis=1)


def _edge_l1_body(he_ref, g_ref, w0_ref, be_ref, out_ref):
    he = he_ref[...][:, :64]
    g = g_ref[...]
    he2 = he + jax.nn.relu(
        _dot(he, w0_ref[...]) + g[:, :64] + g[:, 64:] + be_ref[...])
    out_ref[...] = jnp.concatenate([he2, jnp.zeros_like(he2)], axis=1)


def _node_mlp_proj_body(hn_ref, agg_ref, wn0_ref, wn1_ref, bn_ref,
                        ws_ref, wr_ref, hn1_ref, a_ref, b_ref):
    hn = hn_ref[...]
    hn1 = hn + jax.nn.relu(_dot(hn, wn0_ref[...]) + _dot(agg_ref[...], wn1_ref[...])
                           + bn_ref[...])
    hn1_ref[...] = hn1
    a_ref[...] = _dot(hn1, ws_ref[...])
    b_ref[...] = _dot(hn1, wr_ref[...])


def _node_mlp_dec_body(hn_ref, agg_ref, wn0_ref, wn1_ref, bn_ref,
                       wdec_ref, bdec_ref, nt_ref, bt_ref, y8_ref, sc8_ref,
                       out_ref):
    hn = hn_ref[...]
    hn1 = hn + jax.nn.relu(_dot(hn, wn0_ref[...]) + _dot(agg_ref[...], wn1_ref[...])
                           + bn_ref[...])
    uvp = _dot(hn1, wdec_ref[...]) + bdec_ref[...]
    uvp = jnp.tanh(uvp / 10.0) * 10.0
    nt = nt_ref[...]
    m_d = nt <= 3
    m_p = nt == 2
    uv_part = jnp.where(m_d, y8_ref[...], uvp)
    p_part = jnp.where(m_p, 0.0, uvp)
    li = lax.broadcasted_iota(jnp.int32, uvp.shape, 1)
    res = jnp.where(li < 2, uv_part, p_part)
    bt = bt_ref[...]
    sc8 = sc8_ref[...]
    ssel = jnp.zeros_like(res)
    for b in range(sc8.shape[0]):
        ssel = jnp.where(bt == b, sc8[b], ssel)
    out_ref[...] = res * ssel


def _full(shape):
    return pl.BlockSpec(shape, lambda i: (0, 0))


def _blk(rows, cols):
    return pl.BlockSpec((rows, cols), lambda i: (i, 0))


def _sc_gather_pack(tab_a, tab_b, s_idx, r_idx, D):
    E = s_idx.shape[0]
    C = 1000
    per = E // 16
    nch = per // C
    mesh = plsc.VectorSubcoreMesh(core_axis_name="c", subcore_axis_name="s")

    @functools.partial(
        pl.kernel,
        out_type=jax.ShapeDtypeStruct((E, 2 * D), jnp.float32),
        mesh=mesh,
        scratch_types=[pltpu.VMEM((C,), jnp.int32),
                       pltpu.VMEM((C, D), jnp.float32),
                       pltpu.SemaphoreType.DMA],
        compiler_params=pltpu.CompilerParams(use_tc_tiling_on_sc=False),
    )
    def k(ta, ia, tb, ib, o, iv, rv, sem):
        cid = lax.axis_index("c")
        sid = lax.axis_index("s")

        def run(t, i, col):
            def body(j, _):
                base = sid * per + j * C
                pltpu.sync_copy(i.at[pl.ds(base, C)], iv)
                pltpu.async_copy(t.at[iv], rv, sem).wait()
                pltpu.sync_copy(rv, o.at[pl.ds(base, C), pl.ds(col, D)])
                return 0
            lax.fori_loop(0, nch, body, 0)

        @pl.when(cid == 0)
        def _():
            run(ta, ia, 0)

        @pl.when(cid == 1)
        def _():
            run(tb, ib, D)

    return k(tab_a, s_idx, tab_b, r_idx)


def _sc_segsum(he, r_idx, zeros_half, n_nodes, H):
    E = he.shape[0]
    HH = H // 2
    C = 400
    per = E // 16
    nch = per // C
    rpt = n_nodes // 16
    mesh = plsc.VectorSubcoreMesh(core_axis_name="c", subcore_axis_name="s")

    @functools.partial(
        pl.kernel,
        out_type=jax.ShapeDtypeStruct((n_nodes, H), jnp.float32),
        mesh=mesh,
        scratch_types=[pltpu.VMEM((C,), jnp.int32),
                       pltpu.VMEM((C, HH), jnp.float32),
                       pltpu.VMEM_SHARED((n_nodes, HH), jnp.float32),
                       pltpu.SemaphoreType.DMA],
        compiler_params=pltpu.CompilerParams(use_tc_tiling_on_sc=False),
    )
    def k(he_h, r_h, z_h, out_h, iv, rv, acc, sem):
        cid = lax.axis_index("c")
        sid = lax.axis_index("s")
        pltpu.sync_copy(z_h.at[pl.ds(sid * rpt, rpt)],
                        acc.at[pl.ds(sid * rpt, rpt)])
        plsc.subcore_barrier()

        def body(j, _):
            base = sid * per + j * C
            pltpu.sync_copy(r_h.at[pl.ds(base, C)], iv)
            pltpu.sync_copy(he_h.at[pl.ds(base, C), pl.ds(cid * HH, HH)], rv)
            pltpu.sync_copy(rv, acc.at[iv], add=True)
            return 0
        lax.fori_loop(0, nch, body, 0)
        plsc.subcore_barrier()
        pltpu.sync_copy(acc.at[pl.ds(sid * rpt, rpt)],
                        out_h.at[pl.ds(sid * rpt, rpt), pl.ds(cid * HH, HH)])

    return k(he, r_idx, zeros_half)


def kernel(x, pos, edge_index, batch, node_type, y, uvp_dim, sigma, params):
    N, DIN = x.shape
    E = edge_index.shape[1]
    B = uvp_dim.shape[0]
    H = params["ne"][0].shape[1]
    s = edge_index[0]
    r = edge_index[1]
    bt2 = batch[:, None]
    nt2 = node_type[:, None]
    ng = N // NB
    eg = E // EB
    f32 = jnp.float32

    xmin = pl.pallas_call(
        _xmin_body,
        grid=(ng,),
        in_specs=[_blk(NB, PHI), _blk(NB, 1)],
        out_specs=pl.BlockSpec((B, PHI), lambda i: (0, 0)),
        out_shape=jax.ShapeDtypeStruct((B, PHI), f32),
        scratch_shapes=[pltpu.VMEM((B, PHI), f32)],
    )(x[:, :PHI], bt2)

    xminp = jnp.pad(xmin, ((0, 0), (0, DIN - PHI)))
    shift = jnp.pad(params["norm_mean"], (PHI, 0))[None]
    stdp = jnp.pad(params["norm_std"], (PHI, 0), constant_values=1.0)[None]
    wne, bne = params["ne"][0], params["ne"][1][None]
    wee, bee = params["ee"][0], params["ee"][1][None]
    wp = jnp.pad(wee[:DIN + 2], ((0, 16 - (DIN + 2)), (0, 0)))
    wn_row = wee[DIN + 2:DIN + 3]
    we0 = [params["e%d" % l][0][:H] for l in range(2)]
    wes = [params["e%d" % l][0][H:2 * H] for l in range(2)]
    wer = [params["e%d" % l][0][2 * H:] for l in range(2)]
    bel = [params["e%d" % l][1][None] for l in range(2)]
    wn0 = [params["n%d" % l][0][:H] for l in range(2)]
    wn1 = [params["n%d" % l][0][H:] for l in range(2)]
    bnl = [params["n%d" % l][1][None] for l in range(2)]
    wdec = jnp.pad(params["dec"][0], ((0, 0), (0, 8 - 3)))
    bdec = jnp.pad(params["dec"][1], (0, 8 - 3))[None]
    y8 = jnp.pad(y, ((0, 0), (0, 8 - 3)))
    sc8 = jnp.pad(uvp_dim * sigma, ((0, 0), (0, 8 - 3)))

    t, hn, a, b = pl.pallas_call(
        _node_pre_body,
        grid=(ng,),
        in_specs=[_blk(NB, DIN), _blk(NB, 2), _blk(NB, 1), _full((B, DIN)),
                  _full((1, DIN)), _full((1, DIN)), _full((DIN, H)),
                  _full((1, H)), _full((H, H)), _full((H, H))],
        out_specs=[_blk(NB, 16), _blk(NB, H), _blk(NB, H), _blk(NB, H)],
        out_shape=[jax.ShapeDtypeStruct((N, 16), f32),
                   jax.ShapeDtypeStruct((N, H), f32),
                   jax.ShapeDtypeStruct((N, H), f32),
                   jax.ShapeDtypeStruct((N, H), f32)],
    )(x, pos, bt2, xminp, shift, stdp, wne, bne, wes[0], wer[0])

    tpair = _sc_gather_pack(t, t, s, r, 16)
    g0 = _sc_gather_pack(a, b, s, r, H)

    he = pl.pallas_call(
        _edge_l0_body,
        grid=(eg,),
        in_specs=[_blk(EB, 32), _blk(EB, 2 * H), _full((16, H)),
                  _full((1, H)), _full((1, H)), _full((H, H)), _full((1, H))],
        out_specs=_blk(EB, 2 * H),
        out_shape=jax.ShapeDtypeStruct((E, 2 * H), f32),
    )(tpair, g0, wp, wn_row, bee, we0[0], bel[0])

    zeros_half = jnp.zeros((N, H // 2), f32)
    out = None
    for l in range(2):
        if l == 1:
            g1 = _sc_gather_pack(a, b, s, r, H)
            he = pl.pallas_call(
                _edge_l1_body,
                grid=(eg,),
                in_specs=[_blk(EB, 2 * H), _blk(EB, 2 * H), _full((H, H)),
                          _full((1, H))],
                out_specs=_blk(EB, 2 * H),
                out_shape=jax.ShapeDtypeStruct((E, 2 * H), f32),
            )(he, g1, we0[l], bel[l])
        agg = _sc_segsum(he, r, zeros_half, N, H)
        if l == 0:
            hn, a, b = pl.pallas_call(
                _node_mlp_proj_body,
                grid=(ng,),
                in_specs=[_blk(NB, H), _blk(NB, H), _full((H, H)),
                          _full((H, H)), _full((1, H)), _full((H, H)),
                          _full((H, H))],
                out_specs=[_blk(NB, H), _blk(NB, H), _blk(NB, H)],
                out_shape=[jax.ShapeDtypeStruct((N, H), f32),
                           jax.ShapeDtypeStruct((N, H), f32),
                           jax.ShapeDtypeStruct((N, H), f32)],
            )(hn, agg, wn0[l], wn1[l], bnl[l], wes[1], wer[1])
        else:
            out = pl.pallas_call(
                _node_mlp_dec_body,
                grid=(ng,),
                in_specs=[_blk(NB, H), _blk(NB, H), _full((H, H)),
                          _full((H, H)), _full((1, H)), _full((H, 8)),
                          _full((1, 8)), _blk(NB, 1), _blk(NB, 1),
                          _blk(NB, 8), _full((B, 8))],
                out_specs=_blk(NB, 8),
                out_shape=jax.ShapeDtypeStruct((N, 8), f32),
            )(hn, agg, wn0[l], wn1[l], bnl[l], wdec, bdec, nt2, bt2, y8, sc8)

    return out[:, :3]


# speedup vs baseline: 6.5483x; 1.0465x over previous
import functools

import jax
import jax.numpy as jnp
from jax import lax
from jax.experimental import pallas as pl
from jax.experimental.pallas import tpu as pltpu
from jax.experimental.pallas import tpu_sc as plsc

PHI = 3
NB = 2000
EB = 8000


def _dot(a, b):
    return jax.lax.dot_general(a, b, (((1,), (0,)), ((), ())),
                               preferred_element_type=jnp.float32)


def _xmin_body(xp_ref, bt_ref, out_ref, acc_ref):
    i = pl.program_id(0)
    big = jnp.float32(3.4e38)

    @pl.when(i == 0)
    def _():
        acc_ref[...] = jnp.full(acc_ref.shape, big, jnp.float32)

    xp = xp_ref[...]
    bt = bt_ref[...]
    rows = []
    for b in range(out_ref.shape[0]):
        m = bt == b
        rows.append(jnp.min(jnp.where(m, xp, big), axis=0, keepdims=True))
    acc_ref[...] = jnp.minimum(acc_ref[...], jnp.concatenate(rows, axis=0))

    @pl.when(i == pl.num_programs(0) - 1)
    def _():
        out_ref[...] = acc_ref[...]


def _node_pre_body(x_ref, pos_ref, bt_ref, xminp_ref, shift_ref, stdp_ref,
                   wne_ref, bne_ref, ws_ref, wr_ref,
                   t_ref, hn_ref, a_ref, b_ref):
    x = x_ref[...]
    bt = bt_ref[...]
    xminp = xminp_ref[...]
    sh = jnp.zeros_like(x)
    for b in range(xminp.shape[0]):
        sh = jnp.where(bt == b, xminp[b], sh)
    xn = (x - sh - shift_ref[...]) / stdp_ref[...]
    hn = jax.nn.relu(_dot(xn, wne_ref[...]) + bne_ref[...])
    t_ref[...] = jnp.concatenate(
        [xn, pos_ref[...], jnp.zeros((x.shape[0], 6), jnp.float32)], axis=1)
    hn_ref[...] = hn
    a_ref[...] = _dot(hn, ws_ref[...])
    b_ref[...] = _dot(hn, wr_ref[...])


def _edge_l0_body(tp_ref, g_ref, wp_ref, wn_ref, bee_ref, w0_ref, be_ref,
                  out_ref):
    tp = tp_ref[...]
    d = tp[:, :16] - tp[:, 16:32]
    dx = d[:, 8:9]
    dy = d[:, 9:10]
    nrm = jnp.sqrt(dx * dx + dy * dy)
    he0 = jax.nn.relu(_dot(d, wp_ref[...]) + nrm * wn_ref[...] + bee_ref[...])
    g = g_ref[...]
    he1 = he0 + jax.nn.relu(
        _dot(he0, w0_ref[...]) + g[:, :64] + g[:, 64:] + be_ref[...])
    out_ref[...] = jnp.concatenate([he1, jnp.zeros_like(he1)], axis=1)


def _edge_l1_body(he_ref, g_ref, w0_ref, be_ref, out_ref):
    he = he_ref[...][:, :64]
    g = g_ref[...]
    he2 = he + jax.nn.relu(
        _dot(he, w0_ref[...]) + g[:, :64] + g[:, 64:] + be_ref[...])
    out_ref[...] = jnp.concatenate([he2, jnp.zeros_like(he2)], axis=1)


def _node_mlp_proj_body(hn_ref, agg_ref, wn0_ref, wn1_ref, bn_ref,
                        ws_ref, wr_ref, hn1_ref, a_ref, b_ref):
    hn = hn_ref[...]
    hn1 = hn + jax.nn.relu(_dot(hn, wn0_ref[...]) + _dot(agg_ref[...], wn1_ref[...])
                           + bn_ref[...])
    hn1_ref[...] = hn1
    a_ref[...] = _dot(hn1, ws_ref[...])
    b_ref[...] = _dot(hn1, wr_ref[...])


def _node_mlp_dec_body(hn_ref, agg_ref, wn0_ref, wn1_ref, bn_ref,
                       wdec_ref, bdec_ref, nt_ref, bt_ref, y8_ref, sc8_ref,
                       out_ref):
    hn = hn_ref[...]
    hn1 = hn + jax.nn.relu(_dot(hn, wn0_ref[...]) + _dot(agg_ref[...], wn1_ref[...])
                           + bn_ref[...])
    uvp = _dot(hn1, wdec_ref[...]) + bdec_ref[...]
    uvp = jnp.tanh(uvp / 10.0) * 10.0
    nt = nt_ref[...]
    m_d = nt <= 3
    m_p = nt == 2
    uv_part = jnp.where(m_d, y8_ref[...], uvp)
    p_part = jnp.where(m_p, 0.0, uvp)
    li = lax.broadcasted_iota(jnp.int32, uvp.shape, 1)
    res = jnp.where(li < 2, uv_part, p_part)
    bt = bt_ref[...]
    sc8 = sc8_ref[...]
    ssel = jnp.zeros_like(res)
    for b in range(sc8.shape[0]):
        ssel = jnp.where(bt == b, sc8[b], ssel)
    out_ref[...] = res * ssel


def _full(shape):
    return pl.BlockSpec(shape, lambda i: (0, 0))


def _blk(rows, cols):
    return pl.BlockSpec((rows, cols), lambda i: (i, 0))


def _sc_gather_pack(tab_a, tab_b, s_idx, r_idx, D):
    E = s_idx.shape[0]
    C = 1000
    per = E // 16
    nch = per // C
    W = max(2 * D, 128)
    mesh = plsc.VectorSubcoreMesh(core_axis_name="c", subcore_axis_name="s")

    @functools.partial(
        pl.kernel,
        out_type=jax.ShapeDtypeStruct((E, W), jnp.float32),
        mesh=mesh,
        scratch_types=[pltpu.VMEM((C,), jnp.int32),
                       pltpu.VMEM((C, D), jnp.float32),
                       pltpu.SemaphoreType.DMA],
        compiler_params=pltpu.CompilerParams(use_tc_tiling_on_sc=False),
    )
    def k(ta, ia, tb, ib, o, iv, rv, sem):
        cid = lax.axis_index("c")
        sid = lax.axis_index("s")

        def run(t, i, col):
            def body(j, _):
                base = sid * per + j * C
                pltpu.sync_copy(i.at[pl.ds(base, C)], iv)
                pltpu.async_copy(t.at[iv], rv, sem).wait()
                pltpu.sync_copy(rv, o.at[pl.ds(base, C), pl.ds(col, D)])
                return 0
            lax.fori_loop(0, nch, body, 0)

        @pl.when(cid == 0)
        def _():
            run(ta, ia, 0)

        @pl.when(cid == 1)
        def _():
            run(tb, ib, D)

    return k(tab_a, s_idx, tab_b, r_idx)


def _sc_segsum(he, r_idx, zeros_half, n_nodes, H):
    E = he.shape[0]
    HH = H // 2
    C = 400
    per = E // 16
    nch = per // C
    rpt = n_nodes // 16
    mesh = plsc.VectorSubcoreMesh(core_axis_name="c", subcore_axis_name="s")

    @functools.partial(
        pl.kernel,
        out_type=jax.ShapeDtypeStruct((n_nodes, H), jnp.float32),
        mesh=mesh,
        scratch_types=[pltpu.VMEM((C,), jnp.int32),
                       pltpu.VMEM((C, HH), jnp.float32),
                       pltpu.VMEM_SHARED((n_nodes, HH), jnp.float32),
                       pltpu.SemaphoreType.DMA],
        compiler_params=pltpu.CompilerParams(use_tc_tiling_on_sc=False),
    )
    def k(he_h, r_h, z_h, out_h, iv, rv, acc, sem):
        cid = lax.axis_index("c")
        sid = lax.axis_index("s")
        pltpu.sync_copy(z_h.at[pl.ds(sid * rpt, rpt)],
                        acc.at[pl.ds(sid * rpt, rpt)])
        plsc.subcore_barrier()

        def body(j, _):
            base = sid * per + j * C
            pltpu.sync_copy(r_h.at[pl.ds(base, C)], iv)
            pltpu.sync_copy(he_h.at[pl.ds(base, C), pl.ds(cid * HH, HH)], rv)
            pltpu.sync_copy(rv, acc.at[iv], add=True)
            return 0
        lax.fori_loop(0, nch, body, 0)
        plsc.subcore_barrier()
        pltpu.sync_copy(acc.at[pl.ds(sid * rpt, rpt)],
                        out_h.at[pl.ds(sid * rpt, rpt), pl.ds(cid * HH, HH)])

    return k(he, r_idx, zeros_half)


def kernel(x, pos, edge_index, batch, node_type, y, uvp_dim, sigma, params):
    N, DIN = x.shape
    E = edge_index.shape[1]
    B = uvp_dim.shape[0]
    H = params["ne"][0].shape[1]
    s = edge_index[0]
    r = edge_index[1]
    bt2 = batch[:, None]
    nt2 = node_type[:, None]
    ng = N // NB
    eg = E // EB
    f32 = jnp.float32

    xmin = pl.pallas_call(
        _xmin_body,
        grid=(ng,),
        in_specs=[_blk(NB, PHI), _blk(NB, 1)],
        out_specs=pl.BlockSpec((B, PHI), lambda i: (0, 0)),
        out_shape=jax.ShapeDtypeStruct((B, PHI), f32),
        scratch_shapes=[pltpu.VMEM((B, PHI), f32)],
    )(x[:, :PHI], bt2)

    xminp = jnp.pad(xmin, ((0, 0), (0, DIN - PHI)))
    shift = jnp.pad(params["norm_mean"], (PHI, 0))[None]
    stdp = jnp.pad(params["norm_std"], (PHI, 0), constant_values=1.0)[None]
    wne, bne = params["ne"][0], params["ne"][1][None]
    wee, bee = params["ee"][0], params["ee"][1][None]
    wp = jnp.pad(wee[:DIN + 2], ((0, 16 - (DIN + 2)), (0, 0)))
    wn_row = wee[DIN + 2:DIN + 3]
    we0 = [params["e%d" % l][0][:H] for l in range(2)]
    wes = [params["e%d" % l][0][H:2 * H] for l in range(2)]
    wer = [params["e%d" % l][0][2 * H:] for l in range(2)]
    bel = [params["e%d" % l][1][None] for l in range(2)]
    wn0 = [params["n%d" % l][0][:H] for l in range(2)]
    wn1 = [params["n%d" % l][0][H:] for l in range(2)]
    bnl = [params["n%d" % l][1][None] for l in range(2)]
    wdec = jnp.pad(params["dec"][0], ((0, 0), (0, 8 - 3)))
    bdec = jnp.pad(params["dec"][1], (0, 8 - 3))[None]
    y8 = jnp.pad(y, ((0, 0), (0, 8 - 3)))
    sc8 = jnp.pad(uvp_dim * sigma, ((0, 0), (0, 8 - 3)))

    t, hn, a, b = pl.pallas_call(
        _node_pre_body,
        grid=(ng,),
        in_specs=[_blk(NB, DIN), _blk(NB, 2), _blk(NB, 1), _full((B, DIN)),
                  _full((1, DIN)), _full((1, DIN)), _full((DIN, H)),
                  _full((1, H)), _full((H, H)), _full((H, H))],
        out_specs=[_blk(NB, 16), _blk(NB, H), _blk(NB, H), _blk(NB, H)],
        out_shape=[jax.ShapeDtypeStruct((N, 16), f32),
                   jax.ShapeDtypeStruct((N, H), f32),
                   jax.ShapeDtypeStruct((N, H), f32),
                   jax.ShapeDtypeStruct((N, H), f32)],
    )(x, pos, bt2, xminp, shift, stdp, wne, bne, wes[0], wer[0])

    tpair = _sc_gather_pack(t, t, s, r, 16)
    g0 = _sc_gather_pack(a, b, s, r, H)

    he = pl.pallas_call(
        _edge_l0_body,
        grid=(eg,),
        in_specs=[_blk(EB, 2 * H), _blk(EB, 2 * H), _full((16, H)),
                  _full((1, H)), _full((1, H)), _full((H, H)), _full((1, H))],
        out_specs=_blk(EB, 2 * H),
        out_shape=jax.ShapeDtypeStruct((E, 2 * H), f32),
    )(tpair, g0, wp, wn_row, bee, we0[0], bel[0])

    zeros_half = jnp.zeros((N, H // 2), f32)
    out = None
    for l in range(2):
        if l == 1:
            g1 = _sc_gather_pack(a, b, s, r, H)
            he = pl.pallas_call(
                _edge_l1_body,
                grid=(eg,),
                in_specs=[_blk(EB, 2 * H), _blk(EB, 2 * H), _full((H, H)),
                          _full((1, H))],
                out_specs=_blk(EB, 2 * H),
                out_shape=jax.ShapeDtypeStruct((E, 2 * H), f32),
            )(he, g1, we0[l], bel[l])
        agg = _sc_segsum(he, r, zeros_half, N, H)
        if l == 0:
            hn, a, b = pl.pallas_call(
                _node_mlp_proj_body,
                grid=(ng,),
                in_specs=[_blk(NB, H), _blk(NB, H), _full((H, H)),
                          _full((H, H)), _full((1, H)), _full((H, H)),
                          _full((H, H))],
                out_specs=[_blk(NB, H), _blk(NB, H), _blk(NB, H)],
                out_shape=[jax.ShapeDtypeStruct((N, H), f32),
                           jax.ShapeDtypeStruct((N, H), f32),
                           jax.ShapeDtypeStruct((N, H), f32)],
            )(hn, agg, wn0[l], wn1[l], bnl[l], wes[1], wer[1])
        else:
            out = pl.pallas_call(
                _node_mlp_dec_body,
                grid=(ng,),
                in_specs=[_blk(NB, H), _blk(NB, H), _full((H, H)),
                          _full((H, H)), _full((1, H)), _full((H, 8)),
                          _full((1, 8)), _blk(NB, 1), _blk(NB, 1),
                          _blk(NB, 8), _full((B, 8))],
                out_specs=_blk(NB, 8),
                out_shape=jax.ShapeDtypeStruct((N, 8), f32),
            )(hn, agg, wn0[l], wn1[l], bnl[l], wdec, bdec, nt2, bt2, y8, sc8)

    return out[:, :3]


# speedup vs baseline: 6.5891x; 1.0062x over previous
import functools

import jax
import jax.numpy as jnp
from jax import lax
from jax.experimental import pallas as pl
from jax.experimental.pallas import tpu as pltpu
from jax.experimental.pallas import tpu_sc as plsc

PHI = 3
NB = 2000
EB = 8000


def _dot(a, b):
    return jax.lax.dot_general(a, b, (((1,), (0,)), ((), ())),
                               preferred_element_type=jnp.float32)


def _xmin_body(xp_ref, bt_ref, out_ref, acc_ref):
    i = pl.program_id(0)
    big = jnp.float32(3.4e38)

    @pl.when(i == 0)
    def _():
        acc_ref[...] = jnp.full(acc_ref.shape, big, jnp.float32)

    xp = xp_ref[...]
    bt = bt_ref[...]
    rows = []
    for b in range(out_ref.shape[0]):
        m = bt == b
        rows.append(jnp.min(jnp.where(m, xp, big), axis=0, keepdims=True))
    acc_ref[...] = jnp.minimum(acc_ref[...], jnp.concatenate(rows, axis=0))

    @pl.when(i == pl.num_programs(0) - 1)
    def _():
        out_ref[...] = acc_ref[...]


def _node_pre_body(x_ref, pos_ref, bt_ref, xminp_ref, shift_ref, stdp_ref,
                   wne_ref, bne_ref, ws_ref, wr_ref,
                   t_ref, hn_ref, a_ref, b_ref):
    x = x_ref[...]
    bt = bt_ref[...]
    xminp = xminp_ref[...]
    sh = jnp.zeros_like(x)
    for b in range(xminp.shape[0]):
        sh = jnp.where(bt == b, xminp[b], sh)
    xn = (x - sh - shift_ref[...]) / stdp_ref[...]
    hn = jax.nn.relu(_dot(xn, wne_ref[...]) + bne_ref[...])
    t_ref[...] = jnp.concatenate(
        [xn, pos_ref[...], jnp.zeros((x.shape[0], 6), jnp.float32)], axis=1)
    hn_ref[...] = hn
    a_ref[...] = _dot(hn, ws_ref[...])
    b_ref[...] = _dot(hn, wr_ref[...])


def _edge_l0_body(tp_ref, g_ref, wp_ref, wn_ref, bee_ref, w0_ref, be_ref,
                  out_ref):
    tp = tp_ref[...]
    d = tp[:, :16] - tp[:, 16:32]
    dx = d[:, 8:9]
    dy = d[:, 9:10]
    nrm = jnp.sqrt(dx * dx + dy * dy)
    he0 = jax.nn.relu(_dot(d, wp_ref[...]) + nrm * wn_ref[...] + bee_ref[...])
    g = g_ref[...]
    he1 = he0 + jax.nn.relu(
        _dot(he0, w0_ref[...]) + g[:, :64] + g[:, 64:] + be_ref[...])
    out_ref[...] = jnp.concatenate([he1, jnp.zeros_like(he1)], axis=1)


def _edge_l1_body(he_ref, g_ref, w0_ref, be_ref, out_ref):
    he = he_ref[...][:, :64]
    g = g_ref[...]
    he2 = he + jax.nn.relu(
        _dot(he, w0_ref[...]) + g[:, :64] + g[:, 64:] + be_ref[...])
    out_ref[...] = jnp.concatenate([he2, jnp.zeros_like(he2)], axis=1)


def _node_mlp_proj_body(hn_ref, agga_ref, aggb_ref, wn0_ref, wn1_ref, bn_ref,
                        ws_ref, wr_ref, hn1_ref, a_ref, b_ref):
    hn = hn_ref[...]
    agg = agga_ref[...] + aggb_ref[...]
    hn1 = hn + jax.nn.relu(_dot(hn, wn0_ref[...]) + _dot(agg, wn1_ref[...])
                           + bn_ref[...])
    hn1_ref[...] = hn1
    a_ref[...] = _dot(hn1, ws_ref[...])
    b_ref[...] = _dot(hn1, wr_ref[...])


def _node_mlp_dec_body(hn_ref, agga_ref, aggb_ref, wn0_ref, wn1_ref, bn_ref,
                       wdec_ref, bdec_ref, nt_ref, bt_ref, y8_ref, sc8_ref,
                       out_ref):
    hn = hn_ref[...]
    agg = agga_ref[...] + aggb_ref[...]
    hn1 = hn + jax.nn.relu(_dot(hn, wn0_ref[...]) + _dot(agg, wn1_ref[...])
                           + bn_ref[...])
    uvp = _dot(hn1, wdec_ref[...]) + bdec_ref[...]
    uvp = jnp.tanh(uvp / 10.0) * 10.0
    nt = nt_ref[...]
    m_d = nt <= 3
    m_p = nt == 2
    uv_part = jnp.where(m_d, y8_ref[...], uvp)
    p_part = jnp.where(m_p, 0.0, uvp)
    li = lax.broadcasted_iota(jnp.int32, uvp.shape, 1)
    res = jnp.where(li < 2, uv_part, p_part)
    bt = bt_ref[...]
    sc8 = sc8_ref[...]
    ssel = jnp.zeros_like(res)
    for b in range(sc8.shape[0]):
        ssel = jnp.where(bt == b, sc8[b], ssel)
    out_ref[...] = res * ssel


def _full(shape):
    return pl.BlockSpec(shape, lambda i: (0, 0))


def _blk(rows, cols):
    return pl.BlockSpec((rows, cols), lambda i: (i, 0))


def _sc_gather_pack(tab_a, tab_b, s_idx, r_idx, D):
    E = s_idx.shape[0]
    C = 1000
    per = E // 16
    nch = per // C
    W = max(2 * D, 128)
    mesh = plsc.VectorSubcoreMesh(core_axis_name="c", subcore_axis_name="s")

    @functools.partial(
        pl.kernel,
        out_type=jax.ShapeDtypeStruct((E, W), jnp.float32),
        mesh=mesh,
        scratch_types=[pltpu.VMEM((C,), jnp.int32),
                       pltpu.VMEM((C, D), jnp.float32),
                       pltpu.SemaphoreType.DMA],
        compiler_params=pltpu.CompilerParams(use_tc_tiling_on_sc=False),
    )
    def k(ta, ia, tb, ib, o, iv, rv, sem):
        cid = lax.axis_index("c")
        sid = lax.axis_index("s")

        def run(t, i, col):
            def body(j, _):
                base = sid * per + j * C
                pltpu.sync_copy(i.at[pl.ds(base, C)], iv)
                pltpu.async_copy(t.at[iv], rv, sem).wait()
                pltpu.sync_copy(rv, o.at[pl.ds(base, C), pl.ds(col, D)])
                return 0
            lax.fori_loop(0, nch, body, 0)

        @pl.when(cid == 0)
        def _():
            run(ta, ia, 0)

        @pl.when(cid == 1)
        def _():
            run(tb, ib, D)

    return k(tab_a, s_idx, tab_b, r_idx)


def _sc_segsum(he, r_idx, zeros_half, n_nodes, H):
    E = he.shape[0]
    HH = H // 2
    per = E // 16
    C = 400 if per % 400 == 0 else 200
    nch = per // C
    rpt = n_nodes // 16
    mesh = plsc.VectorSubcoreMesh(core_axis_name="c", subcore_axis_name="s")

    @functools.partial(
        pl.kernel,
        out_type=jax.ShapeDtypeStruct((n_nodes, H), jnp.float32),
        mesh=mesh,
        scratch_types=[pltpu.VMEM((C,), jnp.int32),
                       pltpu.VMEM((C, HH), jnp.float32),
                       pltpu.VMEM_SHARED((n_nodes, HH), jnp.float32),
                       pltpu.SemaphoreType.DMA],
        compiler_params=pltpu.CompilerParams(use_tc_tiling_on_sc=False),
    )
    def k(he_h, r_h, z_h, out_h, iv, rv, acc, sem):
        cid = lax.axis_index("c")
        sid = lax.axis_index("s")
        pltpu.sync_copy(z_h.at[pl.ds(sid * rpt, rpt)],
                        acc.at[pl.ds(sid * rpt, rpt)])
        plsc.subcore_barrier()

        def body(j, _):
            base = sid * per + j * C
            pltpu.sync_copy(r_h.at[pl.ds(base, C)], iv)
            pltpu.sync_copy(he_h.at[pl.ds(base, C), pl.ds(cid * HH, HH)], rv)
            pltpu.sync_copy(rv, acc.at[iv], add=True)
            return 0
        lax.fori_loop(0, nch, body, 0)
        plsc.subcore_barrier()
        pltpu.sync_copy(acc.at[pl.ds(sid * rpt, rpt)],
                        out_h.at[pl.ds(sid * rpt, rpt), pl.ds(cid * HH, HH)])

    return k(he, r_idx, zeros_half)


def kernel(x, pos, edge_index, batch, node_type, y, uvp_dim, sigma, params):
    N, DIN = x.shape
    E = edge_index.shape[1]
    B = uvp_dim.shape[0]
    H = params["ne"][0].shape[1]
    s = edge_index[0]
    r = edge_index[1]
    bt2 = batch[:, None]
    nt2 = node_type[:, None]
    ng = N // NB
    eg = E // EB
    f32 = jnp.float32

    xmin = pl.pallas_call(
        _xmin_body,
        grid=(ng,),
        in_specs=[_blk(NB, PHI), _blk(NB, 1)],
        out_specs=pl.BlockSpec((B, PHI), lambda i: (0, 0)),
        out_shape=jax.ShapeDtypeStruct((B, PHI), f32),
        scratch_shapes=[pltpu.VMEM((B, PHI), f32)],
    )(x[:, :PHI], bt2)

    xminp = jnp.pad(xmin, ((0, 0), (0, DIN - PHI)))
    shift = jnp.pad(params["norm_mean"], (PHI, 0))[None]
    stdp = jnp.pad(params["norm_std"], (PHI, 0), constant_values=1.0)[None]
    wne, bne = params["ne"][0], params["ne"][1][None]
    wee, bee = params["ee"][0], params["ee"][1][None]
    wp = jnp.pad(wee[:DIN + 2], ((0, 16 - (DIN + 2)), (0, 0)))
    wn_row = wee[DIN + 2:DIN + 3]
    we0 = [params["e%d" % l][0][:H] for l in range(2)]
    wes = [params["e%d" % l][0][H:2 * H] for l in range(2)]
    wer = [params["e%d" % l][0][2 * H:] for l in range(2)]
    bel = [params["e%d" % l][1][None] for l in range(2)]
    wn0 = [params["n%d" % l][0][:H] for l in range(2)]
    wn1 = [params["n%d" % l][0][H:] for l in range(2)]
    bnl = [params["n%d" % l][1][None] for l in range(2)]
    wdec = jnp.pad(params["dec"][0], ((0, 0), (0, 8 - 3)))
    bdec = jnp.pad(params["dec"][1], (0, 8 - 3))[None]
    y8 = jnp.pad(y, ((0, 0), (0, 8 - 3)))
    sc8 = jnp.pad(uvp_dim * sigma, ((0, 0), (0, 8 - 3)))

    t, hn, a, b = pl.pallas_call(
        _node_pre_body,
        grid=(ng,),
        in_specs=[_blk(NB, DIN), _blk(NB, 2), _blk(NB, 1), _full((B, DIN)),
                  _full((1, DIN)), _full((1, DIN)), _full((DIN, H)),
                  _full((1, H)), _full((H, H)), _full((H, H))],
        out_specs=[_blk(NB, 16), _blk(NB, H), _blk(NB, H), _blk(NB, H)],
        out_shape=[jax.ShapeDtypeStruct((N, 16), f32),
                   jax.ShapeDtypeStruct((N, H), f32),
                   jax.ShapeDtypeStruct((N, H), f32),
                   jax.ShapeDtypeStruct((N, H), f32)],
    )(x, pos, bt2, xminp, shift, stdp, wne, bne, wes[0], wer[0])

    EH = E // 2
    egh = EH // EB
    sh = [s[:EH], s[EH:]]
    rh = [r[:EH], r[EH:]]
    zeros_half = jnp.zeros((N, H // 2), f32)

    def edge_l0(tp, g):
        return pl.pallas_call(
            _edge_l0_body,
            grid=(egh,),
            in_specs=[_blk(EB, 2 * H), _blk(EB, 2 * H), _full((16, H)),
                      _full((1, H)), _full((1, H)), _full((H, H)),
                      _full((1, H))],
            out_specs=_blk(EB, 2 * H),
            out_shape=jax.ShapeDtypeStruct((EH, 2 * H), f32),
        )(tp, g, wp, wn_row, bee, we0[0], bel[0])

    def edge_l1(he, g):
        return pl.pallas_call(
            _edge_l1_body,
            grid=(egh,),
            in_specs=[_blk(EB, 2 * H), _blk(EB, 2 * H), _full((H, H)),
                      _full((1, H))],
            out_specs=_blk(EB, 2 * H),
            out_shape=jax.ShapeDtypeStruct((EH, 2 * H), f32),
        )(he, g, we0[1], bel[1])

    tp = [_sc_gather_pack(t, t, sh[i], rh[i], 16) for i in range(2)]
    g0 = [_sc_gather_pack(a, b, sh[i], rh[i], H) for i in range(2)]
    he = [edge_l0(tp[i], g0[i]) for i in range(2)]
    agg = [_sc_segsum(he[i], rh[i], zeros_half, N, H) for i in range(2)]

    hn, a, b = pl.pallas_call(
        _node_mlp_proj_body,
        grid=(ng,),
        in_specs=[_blk(NB, H), _blk(NB, H), _blk(NB, H), _full((H, H)),
                  _full((H, H)), _full((1, H)), _full((H, H)),
                  _full((H, H))],
        out_specs=[_blk(NB, H), _blk(NB, H), _blk(NB, H)],
        out_shape=[jax.ShapeDtypeStruct((N, H), f32),
                   jax.ShapeDtypeStruct((N, H), f32),
                   jax.ShapeDtypeStruct((N, H), f32)],
    )(hn, agg[0], agg[1], wn0[0], wn1[0], bnl[0], wes[1], wer[1])

    g1 = [_sc_gather_pack(a, b, sh[i], rh[i], H) for i in range(2)]
    he = [edge_l1(he[i], g1[i]) for i in range(2)]
    agg = [_sc_segsum(he[i], rh[i], zeros_half, N, H) for i in range(2)]

    out = pl.pallas_call(
        _node_mlp_dec_body,
        grid=(ng,),
        in_specs=[_blk(NB, H), _blk(NB, H), _blk(NB, H), _full((H, H)),
                  _full((H, H)), _full((1, H)), _full((H, 8)),
                  _full((1, 8)), _blk(NB, 1), _blk(NB, 1),
                  _blk(NB, 8), _full((B, 8))],
        out_specs=_blk(NB, 8),
        out_shape=jax.ShapeDtypeStruct((N, 8), f32),
    )(hn, agg[0], agg[1], wn0[1], wn1[1], bnl[1], wdec, bdec, nt2, bt2, y8,
      sc8)

    return out[:, :3]
